# Initial kernel scaffold; baseline (speedup 1.0000x reference)
import jax
import jax.numpy as jnp
from jax.experimental import pallas as pl


def kernel(inputs, targets, vocab, emb_in, emb_out):
    raise NotImplementedError("write your pallas kernel here")


# speedup vs baseline: 73.6194x; 73.6194x over previous
import jax
import jax.numpy as jnp
from jax import lax
from jax.experimental import pallas as pl
from jax.experimental.pallas import tpu as pltpu
from jax.experimental.pallas import tpu_sc as plsc

B, T, V, D = 1024, 20, 1000, 64
VP = 1024
TP = 32
NC, NS, L = 2, 16, 16
NW = NC * NS
RW = B // NW


def _scores_kernel(ein_ref, eout_ref, ew_ref):
    w = lax.dot_general(
        ein_ref[...], eout_ref[...],
        dimension_numbers=(((1,), (1,)), ((), ())),
        preferred_element_type=jnp.float32)
    col = lax.broadcasted_iota(jnp.int32, (VP, VP), 1)
    ew_ref[...] = jnp.where(col < V, jnp.exp(w), 0.0)


def _gather_kernel(inp_hbm, ew_hbm, voc_hbm, tgt_hbm,
                   dnm_hbm, sxp_hbm,
                   inp_v, erow_v, voc_v, tgt_v, dnm_v, sxp_v, sem):
    wid = lax.axis_index("s") * NC + lax.axis_index("c")
    base = wid * RW

    pltpu.sync_copy(inp_hbm.at[pl.ds(base, RW)], inp_v)
    gat = pltpu.async_copy(ew_hbm.at[inp_v], erow_v, sem)
    pltpu.sync_copy(voc_hbm.at[pl.ds(base * VP, RW * VP)], voc_v)
    pltpu.sync_copy(tgt_hbm.at[pl.ds(base * TP, RW * TP)], tgt_v)
    gat.wait()

    def row_body(b, carry):
        rowsplat = jnp.full((L,), b, jnp.int32)

        def chunk(j, acc):
            cidx = voc_v[pl.ds(b * VP + j * L, L)]
            return acc + plsc.load_gather(erow_v, [rowsplat, cidx])

        acc = lax.fori_loop(0, VP // L, chunk,
                            jnp.zeros((L,), jnp.float32), unroll=4)
        dnm_v[pl.ds(b * L, L)] = acc

        for j2 in range(TP // L):
            tidx = tgt_v[pl.ds(b * TP + j2 * L, L)]
            sxp_v[pl.ds(b * TP + j2 * L, L)] = plsc.load_gather(
                erow_v, [rowsplat, tidx])
        return carry

    lax.fori_loop(0, RW, row_body, 0)

    pltpu.sync_copy(dnm_v, dnm_hbm.at[pl.ds(base * L, RW * L)])
    pltpu.sync_copy(sxp_v, sxp_hbm.at[pl.ds(base * TP, RW * TP)])


def _loss_kernel(dnm_ref, sxp_ref, out_ref):
    denom = jnp.sum(dnm_ref[...], axis=1, keepdims=True)
    l_denom = jnp.sum(jnp.log(denom))
    col = lax.broadcasted_iota(jnp.int32, (B, TP), 1)
    se = jnp.where(col < T, sxp_ref[...], 1.0)
    l_scores = jnp.sum(jnp.log(se))
    out_ref[...] = jnp.reshape(l_denom / B - l_scores / (B * T), (1, 1))


@jax.jit
def kernel(inputs, targets, vocab, emb_in, emb_out):
    inp = inputs.reshape(B).astype(jnp.int32)
    voc = jnp.pad(vocab.astype(jnp.int32), ((0, 0), (0, VP - V)),
                  constant_values=V).reshape(B * VP)
    tgt = jnp.pad(targets.astype(jnp.int32), ((0, 0), (0, TP - T)),
                  constant_values=V).reshape(B * TP)
    ein = jnp.pad(emb_in, ((0, VP - V), (0, 0)))
    eout = jnp.pad(emb_out, ((0, VP - V), (0, 0)))

    ew = pl.pallas_call(
        _scores_kernel,
        out_shape=jax.ShapeDtypeStruct((VP, VP), jnp.float32),
    )(ein, eout)

    mesh = plsc.VectorSubcoreMesh(core_axis_name="c", subcore_axis_name="s",
                                  num_cores=NC, num_subcores=NS)
    dnm, sxp = pl.kernel(
        _gather_kernel,
        mesh=mesh,
        compiler_params=pltpu.CompilerParams(use_tc_tiling_on_sc=False,
                                             needs_layout_passes=False),
        out_type=[jax.ShapeDtypeStruct((B * L,), jnp.float32),
                  jax.ShapeDtypeStruct((B * TP,), jnp.float32)],
        scratch_types=[
            pltpu.VMEM((RW,), jnp.int32),
            pltpu.VMEM((RW, VP), jnp.float32),
            pltpu.VMEM((RW * VP,), jnp.int32),
            pltpu.VMEM((RW * TP,), jnp.int32),
            pltpu.VMEM((RW * L,), jnp.float32),
            pltpu.VMEM((RW * TP,), jnp.float32),
            pltpu.SemaphoreType.DMA,
        ],
    )(inp, ew, voc, tgt)

    loss = pl.pallas_call(
        _loss_kernel,
        out_shape=jax.ShapeDtypeStruct((1, 1), jnp.float32),
    )(dnm.reshape(B, L), sxp.reshape(B, TP))
    return loss[0, 0]
